# baseline (speedup 1.0000x reference)
import functools

import jax
import jax.numpy as jnp
from jax import lax
from jax.experimental import pallas as pl
from jax.experimental.pallas import tpu as pltpu
from jax.experimental.pallas import tpu_sc as plsc

N = 10000
F = 16
IN_C = 128
HID = 64
K = 10
ALPHA = 0.1
E = 320000

NC, NS = 2, 16
NW = NC * NS
N_ACC = 10112
ROWS_PS = N_ACC // NS
CHUNK = 2560
CHUNKS_PW = 4
EPW = CHUNK * CHUNKS_PW
E_PAD = EPW * NW

_R, _C = 1264, 128

_mesh = plsc.VectorSubcoreMesh(core_axis_name="c", subcore_axis_name="s")


@functools.partial(
    pl.kernel,
    mesh=_mesh,
    compiler_params=pltpu.CompilerParams(use_tc_tiling_on_sc=False),
    out_type=jax.ShapeDtypeStruct((NC * N_ACC, F), jnp.float32),
    scratch_types=[
        pltpu.VMEM((2, CHUNK), jnp.int32),
        pltpu.VMEM((2, CHUNK), jnp.int32),
        pltpu.VMEM((2, CHUNK, F), jnp.float32),
        pltpu.VMEM_SHARED((N_ACC, F), jnp.float32),
        pltpu.SemaphoreType.DMA,
    ],
)
def _gather_scatter(g_hbm, src_hbm, dst_hbm, zero_hbm, out_hbm,
                    sidx, didx, rows, acc, sem):
    c = lax.axis_index("c")
    s = lax.axis_index("s")
    wid = c * NS + s
    r0 = s * ROWS_PS
    base = wid * EPW
    pltpu.sync_copy(zero_hbm.at[pl.ds(r0, ROWS_PS)],
                    acc.at[pl.ds(r0, ROWS_PS)])
    plsc.subcore_barrier()
    pltpu.sync_copy(src_hbm.at[pl.ds(base, CHUNK)], sidx.at[0])
    pltpu.sync_copy(dst_hbm.at[pl.ds(base, CHUNK)], didx.at[0])
    pltpu.async_copy(g_hbm.at[sidx.at[0]], rows.at[0], sem)
    for j in range(CHUNKS_PW):
        b = j % 2
        nb = (j + 1) % 2
        if j + 1 < CHUNKS_PW:
            off = base + (j + 1) * CHUNK
            pltpu.sync_copy(src_hbm.at[pl.ds(off, CHUNK)], sidx.at[nb])
            pltpu.sync_copy(dst_hbm.at[pl.ds(off, CHUNK)], didx.at[nb])
        pltpu.make_async_copy(g_hbm.at[sidx.at[b]], rows.at[b], sem).wait()
        if j + 1 < CHUNKS_PW:
            pltpu.async_copy(g_hbm.at[sidx.at[nb]], rows.at[nb], sem)
        pltpu.sync_copy(rows.at[b], acc.at[didx.at[b]], add=True)
    plsc.subcore_barrier()
    pltpu.sync_copy(acc.at[pl.ds(r0, ROWS_PS)],
                    out_hbm.at[pl.ds(c * N_ACC + r0, ROWS_PS)])


@functools.partial(
    pl.kernel,
    mesh=_mesh,
    compiler_params=pltpu.CompilerParams(use_tc_tiling_on_sc=False),
    out_type=[jax.ShapeDtypeStruct((NC * N_ACC, F), jnp.float32),
              jax.ShapeDtypeStruct((N_ACC, F), jnp.float32)],
    scratch_types=[
        pltpu.VMEM((2, CHUNK), jnp.int32),
        pltpu.VMEM((2, CHUNK), jnp.int32),
        pltpu.VMEM((2, CHUNK, F), jnp.float32),
        pltpu.VMEM_SHARED((N_ACC, F), jnp.float32),
        pltpu.VMEM_SHARED((N_ACC, F), jnp.float32),
        pltpu.SemaphoreType.DMA,
        pltpu.SemaphoreType.DMA,
    ],
)
def _fused_step(p_hbm, g_hbm, x0_hbm, dis_hbm, src_hbm, dst_hbm, zero_hbm,
                pout_hbm, gout_hbm, sidx, didx, rows, gtab, acc, sem, isem):
    c = lax.axis_index("c")
    s = lax.axis_index("s")
    wid = c * NS + s
    r0 = s * ROWS_PS
    base = wid * EPW
    ld = [
        pltpu.async_copy(src_hbm.at[pl.ds(base, CHUNK)], sidx.at[0], sem),
        pltpu.async_copy(dst_hbm.at[pl.ds(base, CHUNK)], didx.at[0], sem),
        pltpu.async_copy(zero_hbm.at[pl.ds(r0, ROWS_PS)],
                         acc.at[pl.ds(r0, ROWS_PS)], sem),
        pltpu.async_copy(p_hbm.at[pl.ds(r0, ROWS_PS)],
                         rows.at[0, pl.ds(0, ROWS_PS)], sem),
        pltpu.async_copy(p_hbm.at[pl.ds(N_ACC + r0, ROWS_PS)],
                         rows.at[0, pl.ds(ROWS_PS, ROWS_PS)], sem),
        pltpu.async_copy(g_hbm.at[pl.ds(r0, ROWS_PS)],
                         rows.at[0, pl.ds(2 * ROWS_PS, ROWS_PS)], sem),
        pltpu.async_copy(x0_hbm.at[pl.ds(r0, ROWS_PS)],
                         rows.at[0, pl.ds(3 * ROWS_PS, ROWS_PS)], sem),
        pltpu.async_copy(dis_hbm.at[pl.ds(r0, ROWS_PS)],
                         rows.at[1, pl.ds(0, ROWS_PS)], sem),
    ]
    for hh in ld:
        hh.wait()

    def _combine(r4, _):
        for u in range(4):
            r = r4 * 4 + u
            p0r = rows[0, r]
            p1r = rows[0, ROWS_PS + r]
            gr = rows[0, 2 * ROWS_PS + r]
            x0r = rows[0, 3 * ROWS_PS + r]
            disr = rows[1, r]
            h = (1.0 - ALPHA) * disr * (p0r + p1r + gr) + ALPHA * x0r
            rows[1, ROWS_PS + r] = h * disr
        return 0

    lax.fori_loop(0, ROWS_PS // 4, _combine, 0)
    gnew = rows.at[1, pl.ds(ROWS_PS, ROWS_PS)]
    pltpu.sync_copy(gnew, gtab.at[pl.ds(r0, ROWS_PS)])

    @pl.when(c == 0)
    def _():
        pltpu.sync_copy(gnew, gout_hbm.at[pl.ds(r0, ROWS_PS)])

    plsc.subcore_barrier()
    pltpu.async_copy(gtab.at[sidx.at[0]], rows.at[0], sem)
    if CHUNKS_PW > 1:
        off = base + CHUNK
        pltpu.async_copy(src_hbm.at[pl.ds(off, CHUNK)], sidx.at[1], isem)
        pltpu.async_copy(dst_hbm.at[pl.ds(off, CHUNK)], didx.at[1], isem)
    for j in range(CHUNKS_PW):
        b = j % 2
        nb = (j + 1) % 2
        if j + 1 < CHUNKS_PW:
            off = base + (j + 1) * CHUNK
            pltpu.make_async_copy(src_hbm.at[pl.ds(off, CHUNK)],
                                  sidx.at[nb], isem).wait()
            pltpu.make_async_copy(dst_hbm.at[pl.ds(off, CHUNK)],
                                  didx.at[nb], isem).wait()
        pltpu.make_async_copy(gtab.at[sidx.at[b]], rows.at[b], sem).wait()
        if j + 1 < CHUNKS_PW:
            pltpu.async_copy(gtab.at[sidx.at[nb]], rows.at[nb], sem)
        pltpu.sync_copy(rows.at[b], acc.at[didx.at[b]], add=True)
        if j + 2 < CHUNKS_PW:
            off2 = base + (j + 2) * CHUNK
            pltpu.async_copy(src_hbm.at[pl.ds(off2, CHUNK)], sidx.at[b], isem)
            pltpu.async_copy(dst_hbm.at[pl.ds(off2, CHUNK)], didx.at[b], isem)
    plsc.subcore_barrier()
    pltpu.sync_copy(acc.at[pl.ds(r0, ROWS_PS)],
                    pout_hbm.at[pl.ds(c * N_ACC + r0, ROWS_PS)])


CHUNK2 = 2048
EPW2 = E_PAD // NS
CHUNKS2 = EPW2 // CHUNK2


@functools.partial(
    pl.kernel,
    mesh=_mesh,
    compiler_params=pltpu.CompilerParams(use_tc_tiling_on_sc=False),
    out_type=jax.ShapeDtypeStruct((NC * N_ACC, F), jnp.float32),
    scratch_types=[
        pltpu.VMEM((2, CHUNK2), jnp.int32),
        pltpu.VMEM((2, CHUNK2), jnp.int32),
        pltpu.VMEM((2, CHUNK2, F), jnp.float32),
        pltpu.VMEM((3, ROWS_PS, F), jnp.float32),
        pltpu.VMEM_SHARED((N_ACC, F), jnp.float32),
        pltpu.VMEM_SHARED((N_ACC, F), jnp.float32),
        pltpu.SemaphoreType.DMA,
        pltpu.SemaphoreType.DMA,
    ],
)
def _appnp_all(g0_hbm, x0_hbm, dis_hbm, src_hbm, dst_hbm, zero_hbm,
               hout_hbm, sidx, didx, rows, loc, gtab, acc, sem, isem):
    c = lax.axis_index("c")
    s = lax.axis_index("s")
    r0 = s * ROWS_PS
    cr0 = c * N_ACC + r0
    ebase = s * EPW2
    ld = [
        pltpu.async_copy(x0_hbm.at[pl.ds(cr0, ROWS_PS)], loc.at[0], sem),
        pltpu.async_copy(dis_hbm.at[pl.ds(r0, ROWS_PS)], loc.at[1], sem),
        pltpu.async_copy(g0_hbm.at[pl.ds(cr0, ROWS_PS)], loc.at[2], sem),
        pltpu.async_copy(zero_hbm.at[pl.ds(r0, ROWS_PS)],
                         acc.at[pl.ds(r0, ROWS_PS)], sem),
    ]
    pltpu.async_copy(src_hbm.at[pl.ds(ebase, CHUNK2)], sidx.at[0], isem)
    pltpu.async_copy(dst_hbm.at[pl.ds(ebase, CHUNK2)], didx.at[0], isem)
    for hh in ld:
        hh.wait()
    pltpu.sync_copy(loc.at[2], gtab.at[pl.ds(r0, ROWS_PS)])
    plsc.subcore_barrier()

    for k in range(1, K + 1):
        pltpu.make_async_copy(src_hbm.at[pl.ds(ebase, CHUNK2)],
                              sidx.at[0], isem).wait()
        pltpu.make_async_copy(dst_hbm.at[pl.ds(ebase, CHUNK2)],
                              didx.at[0], isem).wait()
        pltpu.async_copy(gtab.at[sidx.at[0]], rows.at[0], sem)
        if CHUNKS2 > 1:
            off = ebase + CHUNK2
            pltpu.async_copy(src_hbm.at[pl.ds(off, CHUNK2)], sidx.at[1], isem)
            pltpu.async_copy(dst_hbm.at[pl.ds(off, CHUNK2)], didx.at[1], isem)
        for j in range(CHUNKS2):
            b = j % 2
            nb = (j + 1) % 2
            if j + 1 < CHUNKS2:
                off = ebase + (j + 1) * CHUNK2
                pltpu.make_async_copy(src_hbm.at[pl.ds(off, CHUNK2)],
                                      sidx.at[nb], isem).wait()
                pltpu.make_async_copy(dst_hbm.at[pl.ds(off, CHUNK2)],
                                      didx.at[nb], isem).wait()
            pltpu.make_async_copy(gtab.at[sidx.at[b]], rows.at[b], sem).wait()
            if j + 1 < CHUNKS2:
                pltpu.async_copy(gtab.at[sidx.at[nb]], rows.at[nb], sem)
            pltpu.sync_copy(rows.at[b], acc.at[didx.at[b]], add=True)
            if j + 2 < CHUNKS2:
                off2 = ebase + (j + 2) * CHUNK2
                pltpu.async_copy(src_hbm.at[pl.ds(off2, CHUNK2)],
                                 sidx.at[b], isem)
                pltpu.async_copy(dst_hbm.at[pl.ds(off2, CHUNK2)],
                                 didx.at[b], isem)
        plsc.subcore_barrier()
        pltpu.sync_copy(acc.at[pl.ds(r0, ROWS_PS)],
                        rows.at[0, pl.ds(0, ROWS_PS)])
        zh = None
        if k < K:
            zh = pltpu.async_copy(zero_hbm.at[pl.ds(r0, ROWS_PS)],
                                  acc.at[pl.ds(r0, ROWS_PS)], sem)
            pltpu.async_copy(src_hbm.at[pl.ds(ebase, CHUNK2)],
                             sidx.at[0], isem)
            pltpu.async_copy(dst_hbm.at[pl.ds(ebase, CHUNK2)],
                             didx.at[0], isem)
        last = k == K

        def _combine(r4, _):
            for u in range(4):
                r = r4 * 4 + u
                accr = rows[0, r]
                gr = loc[2, r]
                x0r = loc[0, r]
                disr = loc[1, r]
                h = (1.0 - ALPHA) * disr * (accr + gr) + ALPHA * x0r
                loc[2, r] = h if last else h * disr
            return 0

        lax.fori_loop(0, ROWS_PS // 4, _combine, 0)
        if k < K:
            pltpu.sync_copy(loc.at[2], gtab.at[pl.ds(r0, ROWS_PS)])
            zh.wait()
            plsc.subcore_barrier()
        else:
            pltpu.sync_copy(loc.at[2], hout_hbm.at[pl.ds(cr0, ROWS_PS)])


def _mlp_body(x_ref, w1_ref, b1_ref, w2_ref, b2_ref, o_ref):
    h = lax.dot_general(x_ref[...], w1_ref[...], (((1,), (1,)), ((), ())),
                        preferred_element_type=jnp.float32)
    h = jnp.maximum(h + b1_ref[...], 0.0)
    o_ref[...] = lax.dot_general(h, w2_ref[...], (((1,), (1,)), ((), ())),
                                 preferred_element_type=jnp.float32) + b2_ref[...]


_mlp = pl.pallas_call(
    _mlp_body,
    out_shape=jax.ShapeDtypeStruct((N, F), jnp.float32),
)


def _prep_body(h0_ref, d0_ref, d1_ref, dis_ref, g0_ref):
    dis = lax.rsqrt(d0_ref[...] + d1_ref[...] + 1.0)
    dis_ref[...] = dis
    g0_ref[...] = h0_ref[...] * dis


_prep = pl.pallas_call(
    _prep_body,
    out_shape=[jax.ShapeDtypeStruct((_R, _C), jnp.float32),
               jax.ShapeDtypeStruct((_R, _C), jnp.float32)],
)


def _step_body(p0_ref, p1_ref, g_ref, x0_ref, dis_ref, h_ref, gn_ref):
    dis = dis_ref[...]
    h = (1.0 - ALPHA) * dis * (p0_ref[...] + p1_ref[...] + g_ref[...]) \
        + ALPHA * x0_ref[...]
    h_ref[...] = h
    gn_ref[...] = h * dis


_step = pl.pallas_call(
    _step_body,
    out_shape=[jax.ShapeDtypeStruct((_R, _C), jnp.float32),
               jax.ShapeDtypeStruct((_R, _C), jnp.float32)],
)


def kernel(x, edge_index, training, W1, b1, W2, b2):
    src = edge_index[0]
    dst = edge_index[1]
    npad = E_PAD - E
    ar = jnp.arange(npad, dtype=jnp.int32)
    src_p = jnp.concatenate([src, (ar * 37) % N])
    dst_p = jnp.concatenate([dst, N + ar % (N_ACC - N)])
    zero_acc = jnp.zeros((N_ACC, F), jnp.float32)
    ones_tab = jnp.ones((N_ACC, F), jnp.float32)

    degs = _gather_scatter(ones_tab, src_p, dst_p, zero_acc)
    d0 = degs[0:N_ACC].reshape(_R, _C)
    d1 = degs[N_ACC:].reshape(_R, _C)

    h0 = _mlp(x, W1, b1.reshape(1, HID), W2, b2.reshape(1, F))
    x0 = jnp.pad(h0, ((0, N_ACC - N), (0, 0)))
    x0r = x0.reshape(_R, _C)
    dis_r, g_r = _prep(x0r, d0, d1)

    dis_tab = dis_r.reshape(N_ACC, F)
    g_tab = g_r.reshape(N_ACC, F)
    m0 = (jnp.arange(F) < F // 2).astype(jnp.float32)
    x0h = jnp.concatenate([x0 * m0, x0 * (1.0 - m0)], axis=0)
    g0h = jnp.concatenate([g_tab * m0, g_tab * (1.0 - m0)], axis=0)
    hh = _appnp_all(g0h, x0h, dis_tab, src_p, dst_p, zero_acc)
    return hh[0:N] + hh[N_ACC:N_ACC + N]


# speedup vs baseline: 1.3058x; 1.3058x over previous
import functools

import jax
import jax.numpy as jnp
from jax import lax
from jax.experimental import pallas as pl
from jax.experimental.pallas import tpu as pltpu
from jax.experimental.pallas import tpu_sc as plsc

N = 10000
F = 16
IN_C = 128
HID = 64
K = 10
ALPHA = 0.1
E = 320000

NC, NS = 2, 16
NW = NC * NS
N_ACC = 10112
ROWS_PS = N_ACC // NS
CHUNK = 2560
CHUNKS_PW = 4
EPW = CHUNK * CHUNKS_PW
E_PAD = EPW * NW

_R, _C = 1264, 128

_mesh = plsc.VectorSubcoreMesh(core_axis_name="c", subcore_axis_name="s")


@functools.partial(
    pl.kernel,
    mesh=_mesh,
    compiler_params=pltpu.CompilerParams(use_tc_tiling_on_sc=False),
    out_type=jax.ShapeDtypeStruct((NC * N_ACC, F), jnp.float32),
    scratch_types=[
        pltpu.VMEM((2, CHUNK), jnp.int32),
        pltpu.VMEM((2, CHUNK), jnp.int32),
        pltpu.VMEM((2, CHUNK, F), jnp.float32),
        pltpu.VMEM_SHARED((N_ACC, F), jnp.float32),
        pltpu.SemaphoreType.DMA,
    ],
)
def _gather_scatter(g_hbm, src_hbm, dst_hbm, zero_hbm, out_hbm,
                    sidx, didx, rows, acc, sem):
    c = lax.axis_index("c")
    s = lax.axis_index("s")
    wid = c * NS + s
    r0 = s * ROWS_PS
    base = wid * EPW
    pltpu.sync_copy(zero_hbm.at[pl.ds(r0, ROWS_PS)],
                    acc.at[pl.ds(r0, ROWS_PS)])
    plsc.subcore_barrier()
    pltpu.sync_copy(src_hbm.at[pl.ds(base, CHUNK)], sidx.at[0])
    pltpu.sync_copy(dst_hbm.at[pl.ds(base, CHUNK)], didx.at[0])
    pltpu.async_copy(g_hbm.at[sidx.at[0]], rows.at[0], sem)
    for j in range(CHUNKS_PW):
        b = j % 2
        nb = (j + 1) % 2
        if j + 1 < CHUNKS_PW:
            off = base + (j + 1) * CHUNK
            pltpu.sync_copy(src_hbm.at[pl.ds(off, CHUNK)], sidx.at[nb])
            pltpu.sync_copy(dst_hbm.at[pl.ds(off, CHUNK)], didx.at[nb])
        pltpu.make_async_copy(g_hbm.at[sidx.at[b]], rows.at[b], sem).wait()
        if j + 1 < CHUNKS_PW:
            pltpu.async_copy(g_hbm.at[sidx.at[nb]], rows.at[nb], sem)
        pltpu.sync_copy(rows.at[b], acc.at[didx.at[b]], add=True)
    plsc.subcore_barrier()
    pltpu.sync_copy(acc.at[pl.ds(r0, ROWS_PS)],
                    out_hbm.at[pl.ds(c * N_ACC + r0, ROWS_PS)])


@functools.partial(
    pl.kernel,
    mesh=_mesh,
    compiler_params=pltpu.CompilerParams(use_tc_tiling_on_sc=False),
    out_type=[jax.ShapeDtypeStruct((NC * N_ACC, F), jnp.float32),
              jax.ShapeDtypeStruct((N_ACC, F), jnp.float32)],
    scratch_types=[
        pltpu.VMEM((2, CHUNK), jnp.int32),
        pltpu.VMEM((2, CHUNK), jnp.int32),
        pltpu.VMEM((2, CHUNK, F), jnp.float32),
        pltpu.VMEM_SHARED((N_ACC, F), jnp.float32),
        pltpu.VMEM_SHARED((N_ACC, F), jnp.float32),
        pltpu.SemaphoreType.DMA,
        pltpu.SemaphoreType.DMA,
    ],
)
def _fused_step(p_hbm, g_hbm, x0_hbm, dis_hbm, src_hbm, dst_hbm, zero_hbm,
                pout_hbm, gout_hbm, sidx, didx, rows, gtab, acc, sem, isem):
    c = lax.axis_index("c")
    s = lax.axis_index("s")
    wid = c * NS + s
    r0 = s * ROWS_PS
    base = wid * EPW
    ld = [
        pltpu.async_copy(src_hbm.at[pl.ds(base, CHUNK)], sidx.at[0], sem),
        pltpu.async_copy(dst_hbm.at[pl.ds(base, CHUNK)], didx.at[0], sem),
        pltpu.async_copy(zero_hbm.at[pl.ds(r0, ROWS_PS)],
                         acc.at[pl.ds(r0, ROWS_PS)], sem),
        pltpu.async_copy(p_hbm.at[pl.ds(r0, ROWS_PS)],
                         rows.at[0, pl.ds(0, ROWS_PS)], sem),
        pltpu.async_copy(p_hbm.at[pl.ds(N_ACC + r0, ROWS_PS)],
                         rows.at[0, pl.ds(ROWS_PS, ROWS_PS)], sem),
        pltpu.async_copy(g_hbm.at[pl.ds(r0, ROWS_PS)],
                         rows.at[0, pl.ds(2 * ROWS_PS, ROWS_PS)], sem),
        pltpu.async_copy(x0_hbm.at[pl.ds(r0, ROWS_PS)],
                         rows.at[0, pl.ds(3 * ROWS_PS, ROWS_PS)], sem),
        pltpu.async_copy(dis_hbm.at[pl.ds(r0, ROWS_PS)],
                         rows.at[1, pl.ds(0, ROWS_PS)], sem),
    ]
    for hh in ld:
        hh.wait()

    def _combine(r4, _):
        for u in range(4):
            r = r4 * 4 + u
            p0r = rows[0, r]
            p1r = rows[0, ROWS_PS + r]
            gr = rows[0, 2 * ROWS_PS + r]
            x0r = rows[0, 3 * ROWS_PS + r]
            disr = rows[1, r]
            h = (1.0 - ALPHA) * disr * (p0r + p1r + gr) + ALPHA * x0r
            rows[1, ROWS_PS + r] = h * disr
        return 0

    lax.fori_loop(0, ROWS_PS // 4, _combine, 0)
    gnew = rows.at[1, pl.ds(ROWS_PS, ROWS_PS)]
    pltpu.sync_copy(gnew, gtab.at[pl.ds(r0, ROWS_PS)])

    @pl.when(c == 0)
    def _():
        pltpu.sync_copy(gnew, gout_hbm.at[pl.ds(r0, ROWS_PS)])

    plsc.subcore_barrier()
    pltpu.async_copy(gtab.at[sidx.at[0]], rows.at[0], sem)
    if CHUNKS_PW > 1:
        off = base + CHUNK
        pltpu.async_copy(src_hbm.at[pl.ds(off, CHUNK)], sidx.at[1], isem)
        pltpu.async_copy(dst_hbm.at[pl.ds(off, CHUNK)], didx.at[1], isem)
    for j in range(CHUNKS_PW):
        b = j % 2
        nb = (j + 1) % 2
        if j + 1 < CHUNKS_PW:
            off = base + (j + 1) * CHUNK
            pltpu.make_async_copy(src_hbm.at[pl.ds(off, CHUNK)],
                                  sidx.at[nb], isem).wait()
            pltpu.make_async_copy(dst_hbm.at[pl.ds(off, CHUNK)],
                                  didx.at[nb], isem).wait()
        pltpu.make_async_copy(gtab.at[sidx.at[b]], rows.at[b], sem).wait()
        if j + 1 < CHUNKS_PW:
            pltpu.async_copy(gtab.at[sidx.at[nb]], rows.at[nb], sem)
        pltpu.sync_copy(rows.at[b], acc.at[didx.at[b]], add=True)
        if j + 2 < CHUNKS_PW:
            off2 = base + (j + 2) * CHUNK
            pltpu.async_copy(src_hbm.at[pl.ds(off2, CHUNK)], sidx.at[b], isem)
            pltpu.async_copy(dst_hbm.at[pl.ds(off2, CHUNK)], didx.at[b], isem)
    plsc.subcore_barrier()
    pltpu.sync_copy(acc.at[pl.ds(r0, ROWS_PS)],
                    pout_hbm.at[pl.ds(c * N_ACC + r0, ROWS_PS)])


def _mlp_body(x_ref, w1_ref, b1_ref, w2_ref, b2_ref, o_ref):
    h = lax.dot_general(x_ref[...], w1_ref[...], (((1,), (1,)), ((), ())),
                        preferred_element_type=jnp.float32)
    h = jnp.maximum(h + b1_ref[...], 0.0)
    o_ref[...] = lax.dot_general(h, w2_ref[...], (((1,), (1,)), ((), ())),
                                 preferred_element_type=jnp.float32) + b2_ref[...]


_mlp = pl.pallas_call(
    _mlp_body,
    out_shape=jax.ShapeDtypeStruct((N, F), jnp.float32),
)


def _prep_body(h0_ref, d0_ref, d1_ref, dis_ref, g0_ref):
    dis = lax.rsqrt(d0_ref[...] + d1_ref[...] + 1.0)
    dis_ref[...] = dis
    g0_ref[...] = h0_ref[...] * dis


_prep = pl.pallas_call(
    _prep_body,
    out_shape=[jax.ShapeDtypeStruct((_R, _C), jnp.float32),
               jax.ShapeDtypeStruct((_R, _C), jnp.float32)],
)


def _step_body(p0_ref, p1_ref, g_ref, x0_ref, dis_ref, h_ref, gn_ref):
    dis = dis_ref[...]
    h = (1.0 - ALPHA) * dis * (p0_ref[...] + p1_ref[...] + g_ref[...]) \
        + ALPHA * x0_ref[...]
    h_ref[...] = h
    gn_ref[...] = h * dis


_step = pl.pallas_call(
    _step_body,
    out_shape=[jax.ShapeDtypeStruct((_R, _C), jnp.float32),
               jax.ShapeDtypeStruct((_R, _C), jnp.float32)],
)


def kernel(x, edge_index, training, W1, b1, W2, b2):
    src = edge_index[0]
    dst = edge_index[1]
    npad = E_PAD - E
    ar = jnp.arange(npad, dtype=jnp.int32)
    src_p = jnp.concatenate([src, (ar * 37) % N])
    dst_p = jnp.concatenate([dst, N + ar % (N_ACC - N)])
    zero_acc = jnp.zeros((N_ACC, F), jnp.float32)
    ones_tab = jnp.ones((N_ACC, F), jnp.float32)

    degs = _gather_scatter(ones_tab, src_p, dst_p, zero_acc)
    d0 = degs[0:N_ACC].reshape(_R, _C)
    d1 = degs[N_ACC:].reshape(_R, _C)

    h0 = _mlp(x, W1, b1.reshape(1, HID), W2, b2.reshape(1, F))
    x0 = jnp.pad(h0, ((0, N_ACC - N), (0, 0)))
    x0r = x0.reshape(_R, _C)
    dis_r, g_r = _prep(x0r, d0, d1)

    dis_tab = dis_r.reshape(N_ACC, F)
    g_tab = g_r.reshape(N_ACC, F)
    p = _gather_scatter(g_tab, src_p, dst_p, zero_acc)
    for _ in range(K - 1):
        p, g_tab = _fused_step(p, g_tab, x0, dis_tab, src_p, dst_p, zero_acc)
    p0 = p[0:N_ACC].reshape(_R, _C)
    p1 = p[N_ACC:].reshape(_R, _C)
    h_r, _ = _step(p0, p1, g_tab.reshape(_R, _C), x0r, dis_r)
    return h_r.reshape(N_ACC, F)[:N]
